# baseline (speedup 1.0000x reference)
import functools

import jax
import jax.numpy as jnp
from jax import lax
from jax.experimental import pallas as pl
from jax.experimental.pallas import tpu as pltpu
from jax.experimental.pallas import tpu_sc as plsc

MAX_OBJ = 100
PAD = 112
BASE = 96
SHIFT = 100 - BASE
SLAB = 120
WIN = 24
B = 16
LANES = 16
N_GROUPS = PAD // LANES
C_ALL = 33
LMK_CH = tuple(range(6, 21))

_SEL = {
    0: (0, 0, 0, 0, 0, 0, 0, 1, 1, 1, 1, 1, 2, 2, 2, 2, 2, 3, 3, 3, 3, 3, 3,
        3, 3, 3, 3, 3, 4, 4, 4, 4, 4, 4, 4, 5, 5, 5, 5, 5, 6, 7, 7, 7, 7, 7,
        7, 7, 7, 8, 8, 8, 8, 8, 8, 9, 9, 9, 9, 10, 10, 10, 11, 11, 11, 11,
        11, 11, 11, 11, 11, 12, 12, 12, 12, 12, 12, 12, 12, 12, 12, 12, 13,
        13, 13, 13, 13, 13, 13, 13, 13, 13, 13, 14, 14, 14, 14, 15, 15, 15),
    1: (0, 0, 0, 0, 1, 1, 1, 1, 1, 1, 1, 1, 1, 1, 2, 2, 2, 2, 2, 2, 3, 3, 3,
        3, 3, 4, 4, 4, 4, 4, 4, 4, 4, 4, 4, 4, 5, 5, 6, 6, 6, 6, 6, 6, 6, 6,
        7, 7, 7, 7, 7, 7, 7, 8, 8, 8, 8, 8, 8, 8, 9, 9, 9, 9, 9, 9, 9, 10,
        10, 10, 10, 10, 10, 10, 10, 10, 11, 11, 12, 12, 12, 12, 12, 12, 12,
        13, 13, 13, 13, 13, 14, 14, 14, 14, 14, 14, 15, 15, 15, 15),
    2: (0, 0, 0, 0, 1, 1, 1, 1, 1, 2, 2, 2, 2, 2, 2, 2, 2, 3, 3, 3, 3, 3, 3,
        3, 4, 4, 4, 4, 5, 5, 5, 5, 5, 5, 5, 5, 5, 5, 6, 6, 6, 6, 6, 7, 7, 7,
        7, 7, 8, 8, 8, 8, 8, 9, 9, 9, 9, 9, 9, 10, 10, 10, 10, 10, 10, 10,
        10, 11, 11, 11, 11, 11, 12, 12, 12, 12, 12, 13, 13, 13, 13, 13, 13,
        13, 13, 13, 13, 14, 14, 14, 14, 14, 14, 15, 15, 15, 15, 15, 15, 15),
}

_HEADS = (
    (0, 6, 6, (), 1),
    (16, 12, 21, LMK_CH, 2),
    (8, 0, 6, (), 0),
)


def _const_sorted_vec(lane, vals):
    acc = jnp.full((LANES,), vals[0], jnp.int32)
    for j in range(1, LANES):
        d = vals[j] - vals[j - 1]
        if d:
            acc = acc + jnp.where(lane >= j, jnp.int32(d), jnp.int32(0))
    return acc


def _sc_body(xall, o_head, o_face, o_body, slab, out7, out22):
    wid = lax.axis_index("c") * 16 + lax.axis_index("s")
    outs = (o_head, o_face, o_body)

    for (wid0, ch_off, n_ch, lmks, call_id), out_hbm in zip(_HEADS, outs):
        sel = _SEL[call_id] + (0,) * (PAD - MAX_OBJ)
        n_out = 7 + len(lmks)
        outv = out22 if n_out == 22 else out7
        for g in range(N_GROUPS):
            @pl.when(wid == wid0 + g)
            def _(g=g, ch_off=ch_off, n_ch=n_ch, lmks=lmks, sel=sel,
                  out_hbm=out_hbm, outv=outv):
                pltpu.sync_copy(xall.at[:, pl.ds(g * LANES, WIN), :], slab)
                lane = lax.iota(jnp.int32, LANES)
                bvec = _const_sorted_vec(lane, sel[g * LANES:(g + 1) * LANES])
                rvec = lane + SHIFT
                ch = [plsc.load_gather(
                          slab, [bvec, rvec,
                                 jnp.full((LANES,), ch_off + c, jnp.int32)])
                      for c in range(n_ch)]
                cols = [bvec.astype(jnp.float32),
                        ch[0] - 0.5 * ch[2], ch[1] - 0.5 * ch[3],
                        ch[0] + 0.5 * ch[2], ch[1] + 0.5 * ch[3],
                        jnp.zeros((LANES,), jnp.float32),
                        ch[4] * ch[5]]
                cols += [ch[c] for c in lmks]
                for j, col in enumerate(cols):
                    plsc.store_scatter(
                        outv, [lane, jnp.full((LANES,), j, jnp.int32)], col)
                nrows = LANES if g < N_GROUPS - 1 else MAX_OBJ - g * LANES
                pltpu.sync_copy(outv.at[pl.ds(0, nrows), :],
                                out_hbm.at[pl.ds(g * LANES, nrows), :])


@jax.jit
def kernel(IDetectBody, IDetectHead, IKeypoint):
    sb = lax.slice_in_dim(IDetectBody, BASE, BASE + SLAB, axis=1)
    sh = lax.slice_in_dim(IDetectHead, BASE, BASE + SLAB, axis=1)
    sf = lax.slice_in_dim(IKeypoint, BASE, BASE + SLAB, axis=1)
    xall = jnp.concatenate([sb, sh, sf], axis=2)

    f32 = jnp.float32
    call = pl.kernel(
        _sc_body,
        out_type=(jax.ShapeDtypeStruct((MAX_OBJ, 7), f32),
                  jax.ShapeDtypeStruct((MAX_OBJ, 22), f32),
                  jax.ShapeDtypeStruct((MAX_OBJ, 7), f32)),
        mesh=plsc.VectorSubcoreMesh(core_axis_name="c", subcore_axis_name="s"),
        compiler_params=pltpu.CompilerParams(needs_layout_passes=False,
                                             use_tc_tiling_on_sc=False),
        scratch_types=[
            pltpu.VMEM((B, WIN, C_ALL), f32),
            pltpu.VMEM((LANES, 7), f32),
            pltpu.VMEM((LANES, 22), f32),
        ],
    )
    return call(xall)
